# baseline (speedup 1.0000x reference)
import functools

import jax
import jax.numpy as jnp
from jax import lax
from jax.experimental import pallas as pl
from jax.experimental.pallas import tpu as pltpu
from jax.experimental.pallas import tpu_sc as plsc

EMB = 32
BATCH = 16384
CH = 128
NBUF = 4


def _make_kernel(ngroups, batch):
    info = plsc.get_sparse_core_info()
    nw = info.num_cores * info.num_subcores
    b_per_w = batch // nw
    nch = (2 * b_per_w) // CH
    mesh = plsc.VectorSubcoreMesh(core_axis_name="c", subcore_axis_name="s")

    @functools.partial(
        pl.kernel,
        mesh=mesh,
        out_type=[
            jax.ShapeDtypeStruct((batch // 8, 8, EMB), jnp.float32),
            jax.ShapeDtypeStruct((batch // 8, 8, EMB), jnp.float32),
        ],
        scratch_types=[
            pltpu.VMEM((2 * b_per_w,), jnp.int32),
        ]
        + [pltpu.VMEM((CH // 8, 8, EMB), jnp.float32) for _ in range(NBUF)]
        + [pltpu.SemaphoreType.DMA for _ in range(NBUF)],
        compiler_params=pltpu.CompilerParams(needs_layout_passes=False),
    )
    def gather_kernel(table_hbm, uidx_hbm, iidx_hbm, out_u, out_i,
                      idx_v, *bufs_sems):
        bufs = bufs_sems[:NBUF]
        sems = bufs_sems[NBUF:]
        wid = lax.axis_index("s") * info.num_cores + lax.axis_index("c")
        base = wid * b_per_w
        pltpu.sync_copy(uidx_hbm.at[pl.ds(base, b_per_w)],
                        idx_v.at[pl.ds(0, b_per_w)])
        pltpu.sync_copy(iidx_hbm.at[pl.ds(base, b_per_w)],
                        idx_v.at[pl.ds(b_per_w, b_per_w)])

        def fire(c, buf, sem):
            def blk(kb, _):
                for j in range(16):
                    pltpu.async_copy(
                        table_hbm.at[pl.ds(7, 1), pl.ds(3, 1)],
                        buf.at[pl.ds(kb * 2 + j // 8, 1), pl.ds(j % 8, 1)],
                        sem)
                return 0

            lax.fori_loop(0, CH // 16, blk, 0)

        def wait_and_writeout(c, buf, sem):
            pltpu.make_async_copy(
                table_hbm.at[pl.ds(0, CH // 8)], buf, sem).wait()
            out = out_u if c < nch // 2 else out_i
            grp0 = (base + (c % (nch // 2)) * CH) // 8
            pltpu.sync_copy(buf, out.at[pl.ds(grp0, CH // 8)])

        for c in range(NBUF):
            fire(c, bufs[c], sems[c])
        for c in range(nch):
            p = c % NBUF
            wait_and_writeout(c, bufs[p], sems[p])
            if c + NBUF < nch:
                fire(c + NBUF, bufs[p], sems[p])

    return gather_kernel


def kernel(embeddings, user_ids, item_ids):
    vocab, emb = embeddings.shape
    table3 = embeddings.reshape(vocab // 8, 8, emb)
    batch = user_ids.shape[0]
    fn = _make_kernel(vocab // 8, batch)
    users_emb, items_emb = fn(table3, user_ids, item_ids)
    return (users_emb.reshape(batch, emb), items_emb.reshape(batch, emb))


# speedup vs baseline: 7.2891x; 7.2891x over previous
import functools

import jax
import jax.numpy as jnp
from jax import lax
from jax.experimental import pallas as pl
from jax.experimental.pallas import tpu as pltpu
from jax.experimental.pallas import tpu_sc as plsc

EMB = 32
BATCH = 16384
CH = 128
NBUF = 4


def _make_kernel(ngroups, batch):
    info = plsc.get_sparse_core_info()
    nw = info.num_cores * info.num_subcores
    b_per_w = batch // nw
    nch = (2 * b_per_w) // CH
    mesh = plsc.VectorSubcoreMesh(core_axis_name="c", subcore_axis_name="s")

    @functools.partial(
        pl.kernel,
        mesh=mesh,
        out_type=[
            jax.ShapeDtypeStruct((batch // 8, 8, EMB), jnp.float32),
            jax.ShapeDtypeStruct((batch // 8, 8, EMB), jnp.float32),
        ],
        scratch_types=[
            pltpu.VMEM((2 * b_per_w,), jnp.int32),
        ]
        + [pltpu.VMEM((CH // 8, 8, EMB), jnp.float32) for _ in range(NBUF)]
        + [pltpu.SemaphoreType.DMA for _ in range(NBUF)],
        compiler_params=pltpu.CompilerParams(needs_layout_passes=False),
    )
    def gather_kernel(table_hbm, uidx_hbm, iidx_hbm, out_u, out_i,
                      idx_v, *bufs_sems):
        bufs = bufs_sems[:NBUF]
        sems = bufs_sems[NBUF:]
        wid = lax.axis_index("s") * info.num_cores + lax.axis_index("c")
        base = wid * b_per_w
        pltpu.sync_copy(uidx_hbm.at[pl.ds(base, b_per_w)],
                        idx_v.at[pl.ds(0, b_per_w)])
        pltpu.sync_copy(iidx_hbm.at[pl.ds(base, b_per_w)],
                        idx_v.at[pl.ds(b_per_w, b_per_w)])

        def fire(c, buf, sem):
            def blk(kb, _):
                v = idx_v[pl.ds(c * CH + kb * 16, 16)]
                for j in range(16):
                    pltpu.async_copy(
                        table_hbm.at[pl.ds(v[j] >> 3, 1), pl.ds(v[j] & 7, 1)],
                        buf.at[pl.ds(kb * 2 + j // 8, 1), pl.ds(j % 8, 1)],
                        sem)
                return 0

            lax.fori_loop(0, CH // 16, blk, 0)

        def wait_and_writeout(c, buf, sem):
            pltpu.make_async_copy(
                table_hbm.at[pl.ds(0, CH // 8)], buf, sem).wait()
            out = out_u if c < nch // 2 else out_i
            grp0 = (base + (c % (nch // 2)) * CH) // 8
            pltpu.sync_copy(buf, out.at[pl.ds(grp0, CH // 8)])

        for c in range(NBUF):
            fire(c, bufs[c], sems[c])
        for c in range(nch):
            p = c % NBUF
            wait_and_writeout(c, bufs[p], sems[p])
            if c + NBUF < nch:
                fire(c + NBUF, bufs[p], sems[p])

    return gather_kernel


def kernel(embeddings, user_ids, item_ids):
    vocab, emb = embeddings.shape
    table3 = embeddings.reshape(vocab // 8, 8, emb)
    batch = user_ids.shape[0]
    fn = _make_kernel(vocab // 8, batch)
    users_emb, items_emb = fn(table3, user_ids, item_ids)
    return (users_emb.reshape(batch, emb), items_emb.reshape(batch, emb))
